# baseline (speedup 1.0000x reference)
import functools

import jax
import jax.numpy as jnp
from jax import lax
from jax.experimental import pallas as pl
from jax.experimental.pallas import tpu as pltpu
from jax.experimental.pallas import tpu_sc as plsc


def _argmax_body(x_ref, y_ref, idx_ref, *, lanes, n_classes, d, rows_per_f,
                 y_stride):
    xb = x_ref[0]
    c_iota = lax.broadcasted_iota(jnp.int32, xb.shape, 0)
    m = jnp.max(xb, axis=0, keepdims=True)
    big = jnp.int32(2147483647)
    cand = jnp.where(xb == m, c_iota, big)
    y = jnp.min(cand, axis=0, keepdims=True)
    y_t = jnp.transpose(y, (1, 0))
    y_ref[...] = y_t
    i = pl.program_id(0)
    b = i * lanes + lax.broadcasted_iota(jnp.int32, (lanes, 1), 0)
    f = lax.broadcasted_iota(jnp.int32, (1, d), 1)
    idx_ref[...] = (f * rows_per_f + (y_t >> 3) * (8 * y_stride)
                    + (b >> 7) * 8 + (y_t & 7))


def _expand_body(y_ref, g_ref, *rest, bb, d):
    out_ref = rest[-1]
    cols = out_ref.shape[1]
    y = y_ref[...]
    g = g_ref[...]
    bi = lax.broadcasted_iota(jnp.int32, g.shape, 0)
    li = lax.broadcasted_iota(jnp.int32, g.shape, 2)
    vals = jnp.sum(jnp.where(li == (bi & 127), g, jnp.float32(0.0)), axis=2)
    v128 = jnp.concatenate([vals] * (128 // d), axis=1)
    vfull = jnp.concatenate([v128] * (cols // 128), axis=1)
    j = lax.broadcasted_iota(jnp.int32, (bb, cols), 1)
    mask = (j >> 4) == y
    out_ref[...] = jnp.where(mask, vfull, jnp.float32(0.0))


def _make_sc_gather(n_idx, b_per_w, nc, ns):
    mesh = plsc.VectorSubcoreMesh(core_axis_name="c", subcore_axis_name="s")
    per_w = n_idx // (nc * ns)
    n_chunks = per_w // 128

    @functools.partial(
        pl.kernel,
        mesh=mesh,
        out_type=jax.ShapeDtypeStruct((n_idx, 128), jnp.float32),
        scratch_types=[
            pltpu.VMEM((per_w,), jnp.int32),
            pltpu.VMEM((128, 128), jnp.float32),
            pltpu.VMEM((128, 128), jnp.float32),
            pltpu.SemaphoreType.DMA,
            pltpu.SemaphoreType.DMA,
        ],
    )
    def gather_k(table_hbm, idx_hbm, out_hbm, idx_v, buf0, buf1, sem0, sem1):
        wid = lax.axis_index("s") * nc + lax.axis_index("c")
        base = wid * per_w
        pltpu.sync_copy(idx_hbm.at[pl.ds(base, per_w)], idx_v)
        bufs = (buf0, buf1)
        sems = (sem0, sem1)
        cps = [None, None]
        cps[0] = pltpu.async_copy(
            table_hbm.at[idx_v.at[pl.ds(0, 128)]], buf0, sem0)
        for k in range(n_chunks):
            if k + 1 < n_chunks:
                cps[(k + 1) % 2] = pltpu.async_copy(
                    table_hbm.at[idx_v.at[pl.ds((k + 1) * 128, 128)]],
                    bufs[(k + 1) % 2], sems[(k + 1) % 2])
            cps[k % 2].wait()
            pltpu.sync_copy(bufs[k % 2], out_hbm.at[pl.ds(base + k * 128, 128)])

    return gather_k


def kernel(x):
    b, n_classes, d1 = x.shape
    d = d1 - 1
    xt = jnp.transpose(x, (2, 1, 0))
    table_rows = (b * n_classes * d1) // 128
    rows_per_f = (n_classes * b) // 128

    lanes = 512
    y2, idx_list = pl.pallas_call(
        functools.partial(_argmax_body, lanes=lanes, n_classes=n_classes,
                          d=d, rows_per_f=rows_per_f, y_stride=b // 128),
        grid=(b // lanes,),
        in_specs=[pl.BlockSpec((1, n_classes, lanes), lambda i: (d, 0, i))],
        out_specs=[
            pl.BlockSpec((lanes, 1), lambda i: (i, 0)),
            pl.BlockSpec((lanes, d), lambda i: (i, 0)),
        ],
        out_shape=[
            jax.ShapeDtypeStruct((b, 1), jnp.int32),
            jax.ShapeDtypeStruct((b, d), jnp.int32),
        ],
    )(xt)

    info = plsc.get_sparse_core_info()
    nw = info.num_cores * info.num_subcores
    table = (xt.reshape(d1, n_classes // 8, 8, b // 128, 128)
             .transpose(0, 1, 3, 2, 4)
             .reshape(table_rows, 128))

    b0 = b // 4
    b1 = b - b0
    gk0 = _make_sc_gather(b0 * d, (b0 // nw) * d, info.num_cores,
                          info.num_subcores)
    gk1 = _make_sc_gather(b1 * d, (b1 // nw) * d, info.num_cores,
                          info.num_subcores)
    g0 = gk0(table, idx_list[:b0].reshape(b0 * d)).reshape(b0, d, 128)
    g1 = gk1(table, idx_list[b0:].reshape(b1 * d)).reshape(b1, d, 128)

    bb = 256
    cols = n_classes * d
    out_shape = jax.ShapeDtypeStruct((b, cols), jnp.float32)
    out0 = pl.pallas_call(
        functools.partial(_expand_body, bb=bb, d=d),
        grid=(b0 // bb,),
        in_specs=[
            pl.BlockSpec((bb, 1), lambda i: (i, 0)),
            pl.BlockSpec((bb, d, 128), lambda i: (i, 0, 0)),
        ],
        out_specs=pl.BlockSpec((bb, cols), lambda i: (i, 0)),
        out_shape=out_shape,
    )(y2, g0)
    nskip = b0 // bb
    out = pl.pallas_call(
        functools.partial(_expand_body, bb=bb, d=d),
        grid=(b1 // bb,),
        in_specs=[
            pl.BlockSpec((bb, 1), lambda i: (i + nskip, 0)),
            pl.BlockSpec((bb, d, 128), lambda i: (i, 0, 0)),
            pl.BlockSpec(memory_space=pl.ANY),
        ],
        out_specs=pl.BlockSpec((bb, cols), lambda i: (i + nskip, 0)),
        out_shape=out_shape,
        input_output_aliases={2: 0},
    )(y2, g1, out0)
    return out


# speedup vs baseline: 1.0138x; 1.0138x over previous
import functools

import jax
import jax.numpy as jnp
from jax import lax
from jax.experimental import pallas as pl
from jax.experimental.pallas import tpu as pltpu
from jax.experimental.pallas import tpu_sc as plsc


def _argmax_body(x_ref, y_ref, idx_ref, *, lanes, n_classes, d, rows_per_f,
                 y_stride):
    xb = x_ref[0]
    c_iota = lax.broadcasted_iota(jnp.int32, xb.shape, 0)
    m = jnp.max(xb, axis=0, keepdims=True)
    big = jnp.int32(2147483647)
    cand = jnp.where(xb == m, c_iota, big)
    y = jnp.min(cand, axis=0, keepdims=True)
    y_t = jnp.transpose(y, (1, 0))
    y_ref[...] = y_t
    i = pl.program_id(0)
    b = i * lanes + lax.broadcasted_iota(jnp.int32, (lanes, 1), 0)
    f = lax.broadcasted_iota(jnp.int32, (1, d), 1)
    idx_ref[...] = (f * rows_per_f + (y_t >> 3) * (8 * y_stride)
                    + (b >> 7) * 8 + (y_t & 7))


def _expand_body(y_ref, g_ref, *rest, bb, d):
    out_ref = rest[-1]
    cols = out_ref.shape[1]
    y = y_ref[...]
    g = g_ref[...]
    bi = lax.broadcasted_iota(jnp.int32, g.shape, 0)
    li = lax.broadcasted_iota(jnp.int32, g.shape, 2)
    vals = jnp.sum(jnp.where(li == (bi & 127), g, jnp.float32(0.0)), axis=2)
    v128 = jnp.concatenate([vals] * (128 // d), axis=1)
    vfull = jnp.concatenate([v128] * (cols // 128), axis=1)
    j = lax.broadcasted_iota(jnp.int32, (bb, cols), 1)
    mask = (j >> 4) == y
    out_ref[...] = jnp.where(mask, vfull, jnp.float32(0.0))


def _make_sc_gather(n_idx, b_per_w, nc, ns):
    mesh = plsc.VectorSubcoreMesh(core_axis_name="c", subcore_axis_name="s")
    per_w = n_idx // (nc * ns)
    n_chunks = per_w // 128

    nbuf = 4

    @functools.partial(
        pl.kernel,
        mesh=mesh,
        out_type=jax.ShapeDtypeStruct((n_idx, 128), jnp.float32),
        scratch_types=[
            pltpu.VMEM((per_w,), jnp.int32),
        ]
        + [pltpu.VMEM((128, 128), jnp.float32) for _ in range(nbuf)]
        + [pltpu.SemaphoreType.DMA for _ in range(2 * nbuf)],
    )
    def gather_k(table_hbm, idx_hbm, out_hbm, idx_v, *rest):
        bufs = rest[:nbuf]
        gsems = rest[nbuf:2 * nbuf]
        csems = rest[2 * nbuf:]
        wid = lax.axis_index("s") * nc + lax.axis_index("c")
        base = wid * per_w
        pltpu.sync_copy(idx_hbm.at[pl.ds(base, per_w)], idx_v)

        def fire_gather(k):
            return pltpu.async_copy(
                table_hbm.at[idx_v.at[pl.ds(k * 128, 128)]],
                bufs[k % nbuf], gsems[k % nbuf])

        gcps = {}
        ccps = {}
        for k in range(min(nbuf, n_chunks)):
            gcps[k] = fire_gather(k)
        for k in range(n_chunks):
            gcps.pop(k).wait()
            ccps[k] = pltpu.async_copy(
                bufs[k % nbuf], out_hbm.at[pl.ds(base + k * 128, 128)],
                csems[k % nbuf])
            nxt = k + nbuf
            if nxt < n_chunks:
                ccps.pop(k).wait()
                gcps[nxt] = fire_gather(nxt)
        for k in sorted(ccps):
            ccps[k].wait()

    return gather_k


def kernel(x):
    b, n_classes, d1 = x.shape
    d = d1 - 1
    xt = jnp.transpose(x, (2, 1, 0))
    table_rows = (b * n_classes * d1) // 128
    rows_per_f = (n_classes * b) // 128

    lanes = 512
    y2, idx_list = pl.pallas_call(
        functools.partial(_argmax_body, lanes=lanes, n_classes=n_classes,
                          d=d, rows_per_f=rows_per_f, y_stride=b // 128),
        grid=(b // lanes,),
        in_specs=[pl.BlockSpec((1, n_classes, lanes), lambda i: (d, 0, i))],
        out_specs=[
            pl.BlockSpec((lanes, 1), lambda i: (i, 0)),
            pl.BlockSpec((lanes, d), lambda i: (i, 0)),
        ],
        out_shape=[
            jax.ShapeDtypeStruct((b, 1), jnp.int32),
            jax.ShapeDtypeStruct((b, d), jnp.int32),
        ],
    )(xt)

    info = plsc.get_sparse_core_info()
    nw = info.num_cores * info.num_subcores
    table = (xt.reshape(d1, n_classes // 8, 8, b // 128, 128)
             .transpose(0, 1, 3, 2, 4)
             .reshape(table_rows, 128))

    b0 = b // 4
    b1 = b - b0
    gk0 = _make_sc_gather(b0 * d, (b0 // nw) * d, info.num_cores,
                          info.num_subcores)
    gk1 = _make_sc_gather(b1 * d, (b1 // nw) * d, info.num_cores,
                          info.num_subcores)
    g0 = gk0(table, idx_list[:b0].reshape(b0 * d)).reshape(b0, d, 128)
    g1 = gk1(table, idx_list[b0:].reshape(b1 * d)).reshape(b1, d, 128)

    bb0 = 128
    bb = 256
    cols = n_classes * d
    out_shape = jax.ShapeDtypeStruct((b, cols), jnp.float32)
    out0 = pl.pallas_call(
        functools.partial(_expand_body, bb=bb0, d=d),
        grid=(b0 // bb0,),
        in_specs=[
            pl.BlockSpec((bb0, 1), lambda i: (i, 0)),
            pl.BlockSpec((bb0, d, 128), lambda i: (i, 0, 0)),
        ],
        out_specs=pl.BlockSpec((bb0, cols), lambda i: (i, 0)),
        out_shape=out_shape,
    )(y2, g0)
    nskip = b0 // bb
    out = pl.pallas_call(
        functools.partial(_expand_body, bb=bb, d=d),
        grid=(b1 // bb,),
        in_specs=[
            pl.BlockSpec((bb, 1), lambda i: (i + nskip, 0)),
            pl.BlockSpec((bb, d, 128), lambda i: (i, 0, 0)),
            pl.BlockSpec(memory_space=pl.ANY),
        ],
        out_specs=pl.BlockSpec((bb, cols), lambda i: (i + nskip, 0)),
        out_shape=out_shape,
        input_output_aliases={2: 0},
    )(y2, g1, out0)
    return out


# speedup vs baseline: 1.0285x; 1.0145x over previous
import functools

import jax
import jax.numpy as jnp
from jax import lax
from jax.experimental import pallas as pl
from jax.experimental.pallas import tpu as pltpu
from jax.experimental.pallas import tpu_sc as plsc


def _argmax_body(x_ref, y_ref, idx_ref, *, lanes, n_classes, d, rows_per_f,
                 y_stride):
    xb = x_ref[0]
    c_iota = lax.broadcasted_iota(jnp.int32, xb.shape, 0)
    m = jnp.max(xb, axis=0, keepdims=True)
    big = jnp.int32(2147483647)
    cand = jnp.where(xb == m, c_iota, big)
    y = jnp.min(cand, axis=0, keepdims=True)
    y_t = jnp.transpose(y, (1, 0))
    y_ref[...] = y_t
    i = pl.program_id(0)
    b = i * lanes + lax.broadcasted_iota(jnp.int32, (lanes, 1), 0)
    f = lax.broadcasted_iota(jnp.int32, (1, d), 1)
    idx_ref[...] = (f * rows_per_f + (y_t >> 3) * (8 * y_stride)
                    + (b >> 7) * 8 + (y_t & 7))


def _expand_body(y_ref, g_ref, *rest, bb, d):
    out_ref = rest[-1]
    cols = out_ref.shape[1]
    y = y_ref[...]
    g = g_ref[...]
    bi = lax.broadcasted_iota(jnp.int32, g.shape, 0)
    li = lax.broadcasted_iota(jnp.int32, g.shape, 2)
    vals = jnp.sum(jnp.where(li == (bi & 127), g, jnp.float32(0.0)), axis=2)
    v128 = jnp.concatenate([vals] * (128 // d), axis=1)
    vfull = jnp.concatenate([v128] * (cols // 128), axis=1)
    j = lax.broadcasted_iota(jnp.int32, (bb, cols), 1)
    mask = (j >> 4) == y
    out_ref[...] = jnp.where(mask, vfull, jnp.float32(0.0))


def _make_sc_gather(n_idx, b_per_w, nc, ns):
    mesh = plsc.VectorSubcoreMesh(core_axis_name="c", subcore_axis_name="s")
    per_w = n_idx // (nc * ns)
    n_chunks = per_w // 128

    nbuf = 4

    @functools.partial(
        pl.kernel,
        mesh=mesh,
        out_type=jax.ShapeDtypeStruct((n_idx, 128), jnp.float32),
        scratch_types=[
            pltpu.VMEM((per_w,), jnp.int32),
        ]
        + [pltpu.VMEM((128, 128), jnp.float32) for _ in range(nbuf)]
        + [pltpu.SemaphoreType.DMA for _ in range(2 * nbuf)],
    )
    def gather_k(table_hbm, idx_hbm, out_hbm, idx_v, *rest):
        bufs = rest[:nbuf]
        gsems = rest[nbuf:2 * nbuf]
        csems = rest[2 * nbuf:]
        wid = lax.axis_index("s") * nc + lax.axis_index("c")
        base = wid * per_w
        pltpu.sync_copy(idx_hbm.at[pl.ds(base, per_w)], idx_v)

        def fire_gather(k):
            return pltpu.async_copy(
                table_hbm.at[idx_v.at[pl.ds(k * 128, 128)]],
                bufs[k % nbuf], gsems[k % nbuf])

        gcps = {}
        ccps = {}
        for k in range(min(nbuf, n_chunks)):
            gcps[k] = fire_gather(k)
        for k in range(n_chunks):
            gcps.pop(k).wait()
            ccps[k] = pltpu.async_copy(
                bufs[k % nbuf], out_hbm.at[pl.ds(base + k * 128, 128)],
                csems[k % nbuf])
            nxt = k + nbuf
            if nxt < n_chunks:
                ccps.pop(k).wait()
                gcps[nxt] = fire_gather(nxt)
        for k in sorted(ccps):
            ccps[k].wait()

    return gather_k


def kernel(x):
    b, n_classes, d1 = x.shape
    d = d1 - 1
    xt = jnp.transpose(x, (2, 1, 0))
    table_rows = (b * n_classes * d1) // 128
    rows_per_f = (n_classes * b) // 128

    lanes = 512
    y2, idx_list = pl.pallas_call(
        functools.partial(_argmax_body, lanes=lanes, n_classes=n_classes,
                          d=d, rows_per_f=rows_per_f, y_stride=b // 128),
        grid=(b // lanes,),
        in_specs=[pl.BlockSpec((1, n_classes, lanes), lambda i: (d, 0, i))],
        out_specs=[
            pl.BlockSpec((lanes, 1), lambda i: (i, 0)),
            pl.BlockSpec((lanes, d), lambda i: (i, 0)),
        ],
        out_shape=[
            jax.ShapeDtypeStruct((b, 1), jnp.int32),
            jax.ShapeDtypeStruct((b, d), jnp.int32),
        ],
    )(xt)

    info = plsc.get_sparse_core_info()
    nw = info.num_cores * info.num_subcores
    table = (xt.reshape(d1, n_classes // 8, 8, b // 128, 128)
             .transpose(0, 1, 3, 2, 4)
             .reshape(table_rows, 128))

    gather_k = _make_sc_gather(b * d, (b // nw) * d, info.num_cores,
                               info.num_subcores)
    g3 = gather_k(table, idx_list.reshape(b * d)).reshape(b, d, 128)

    bb = 256
    cols = n_classes * d
    out = pl.pallas_call(
        functools.partial(_expand_body, bb=bb, d=d),
        grid=(b // bb,),
        in_specs=[
            pl.BlockSpec((bb, 1), lambda i: (i, 0)),
            pl.BlockSpec((bb, d, 128), lambda i: (i, 0, 0)),
        ],
        out_specs=pl.BlockSpec((bb, cols), lambda i: (i, 0)),
        out_shape=jax.ShapeDtypeStruct((b, cols), jnp.float32),
    )(y2, g3)
    return out
